# baseline (speedup 1.0000x reference)
import jax
import jax.numpy as jnp
from jax.experimental import pallas as pl

_BN = 2000
_KP = 512
_OFF_CLS = 0
_OFF_BBOX = 128
_OFF_IOU = 448


def _heads_kernel(x_ref, w_ref, bc_ref, bb_ref, bi_ref, s_ref, d_ref, i_ref):
    xb = x_ref[...].astype(jnp.bfloat16)
    y = jnp.dot(xb, w_ref[...], preferred_element_type=jnp.float32)
    kc = s_ref.shape[1]
    kb = d_ref.shape[1]
    ki = i_ref.shape[1]
    s_ref[...] = y[:, _OFF_CLS:_OFF_CLS + kc] + bc_ref[...]
    d_ref[...] = y[:, _OFF_BBOX:_OFF_BBOX + kb] + bb_ref[...]
    i_ref[...] = y[:, _OFF_IOU:_OFF_IOU + ki] + bi_ref[...]


def kernel(x, W_cls, b_cls, W_bbox, b_bbox, W_iou, b_iou):
    if x.ndim > 2:
        x = x.reshape(x.shape[0], -1)
    n, d = x.shape
    kc = W_cls.shape[0]
    kb = W_bbox.shape[0]
    ki = W_iou.shape[0]

    w = jnp.zeros((d, _KP), dtype=jnp.bfloat16)
    w = w.at[:, _OFF_CLS:_OFF_CLS + kc].set(W_cls.T.astype(jnp.bfloat16))
    w = w.at[:, _OFF_BBOX:_OFF_BBOX + kb].set(W_bbox.T.astype(jnp.bfloat16))
    w = w.at[:, _OFF_IOU:_OFF_IOU + ki].set(W_iou.T.astype(jnp.bfloat16))
    bc = b_cls.reshape(1, kc)
    bb = b_bbox.reshape(1, kb)
    bi = b_iou.reshape(1, ki)

    grid = (n // _BN,)
    row_block = lambda i: (i, 0)
    whole = lambda i: (0, 0)

    scores, deltas, iou = pl.pallas_call(
        _heads_kernel,
        grid=grid,
        in_specs=[
            pl.BlockSpec((_BN, d), row_block),
            pl.BlockSpec((d, _KP), whole),
            pl.BlockSpec((1, kc), whole),
            pl.BlockSpec((1, kb), whole),
            pl.BlockSpec((1, ki), whole),
        ],
        out_specs=[
            pl.BlockSpec((_BN, kc), row_block),
            pl.BlockSpec((_BN, kb), row_block),
            pl.BlockSpec((_BN, ki), row_block),
        ],
        out_shape=[
            jax.ShapeDtypeStruct((n, kc), jnp.float32),
            jax.ShapeDtypeStruct((n, kb), jnp.float32),
            jax.ShapeDtypeStruct((n, ki), jnp.float32),
        ],
    )(x, w, bc, bb, bi)
    return scores, deltas, iou


# speedup vs baseline: 1.0832x; 1.0832x over previous
import jax
import jax.numpy as jnp
from jax.experimental import pallas as pl

_BN = 2000
_KP = 512
_OFF_CLS = 0
_OFF_BBOX = 128
_OFF_IOU = 448


_NSPLIT = 4


def _heads_kernel(x0_ref, x1_ref, x2_ref, x3_ref, w_ref,
                  bc_ref, bb_ref, bi_ref, s_ref, d_ref, i_ref):
    dk = x0_ref.shape[1]
    y = jnp.dot(x0_ref[...].astype(jnp.bfloat16), w_ref[0:dk, :],
                preferred_element_type=jnp.float32)
    y += jnp.dot(x1_ref[...].astype(jnp.bfloat16), w_ref[dk:2 * dk, :],
                 preferred_element_type=jnp.float32)
    y += jnp.dot(x2_ref[...].astype(jnp.bfloat16), w_ref[2 * dk:3 * dk, :],
                 preferred_element_type=jnp.float32)
    y += jnp.dot(x3_ref[...].astype(jnp.bfloat16), w_ref[3 * dk:4 * dk, :],
                 preferred_element_type=jnp.float32)
    kc = s_ref.shape[1]
    kb = d_ref.shape[1]
    ki = i_ref.shape[1]
    s_ref[...] = y[:, _OFF_CLS:_OFF_CLS + kc] + bc_ref[...]
    d_ref[...] = y[:, _OFF_BBOX:_OFF_BBOX + kb] + bb_ref[...]
    i_ref[...] = y[:, _OFF_IOU:_OFF_IOU + ki] + bi_ref[...]


def kernel(x, W_cls, b_cls, W_bbox, b_bbox, W_iou, b_iou):
    if x.ndim > 2:
        x = x.reshape(x.shape[0], -1)
    n, d = x.shape
    kc = W_cls.shape[0]
    kb = W_bbox.shape[0]
    ki = W_iou.shape[0]

    w = jnp.concatenate([
        W_cls.T, jnp.zeros((d, _OFF_BBOX - kc), jnp.float32),
        W_bbox.T, W_iou.T,
        jnp.zeros((d, _KP - _OFF_IOU - ki), jnp.float32),
    ], axis=1).astype(jnp.bfloat16)
    bc = b_cls.reshape(1, kc)
    bb = b_bbox.reshape(1, kb)
    bi = b_iou.reshape(1, ki)

    grid = (n // _BN,)
    row_block = lambda i: (i, 0)
    whole = lambda i: (0, 0)
    dk = d // _NSPLIT

    def col_chunk(j):
        return pl.BlockSpec((_BN, dk), lambda i, j=j: (i, j))

    scores, deltas, iou = pl.pallas_call(
        _heads_kernel,
        grid=grid,
        in_specs=[
            col_chunk(0), col_chunk(1), col_chunk(2), col_chunk(3),
            pl.BlockSpec((d, _KP), whole),
            pl.BlockSpec((1, kc), whole),
            pl.BlockSpec((1, kb), whole),
            pl.BlockSpec((1, ki), whole),
        ],
        out_specs=[
            pl.BlockSpec((_BN, kc), row_block),
            pl.BlockSpec((_BN, kb), row_block),
            pl.BlockSpec((_BN, ki), row_block),
        ],
        out_shape=[
            jax.ShapeDtypeStruct((n, kc), jnp.float32),
            jax.ShapeDtypeStruct((n, kb), jnp.float32),
            jax.ShapeDtypeStruct((n, ki), jnp.float32),
        ],
    )(x, x, x, x, w, bc, bb, bi)
    return scores, deltas, iou
